# Initial kernel scaffold; baseline (speedup 1.0000x reference)
import jax
import jax.numpy as jnp
from jax.experimental import pallas as pl


def kernel(xyz, W1, g1, b1, W2, g2, b2):
    raise NotImplementedError("write your pallas kernel here")


# speedup vs baseline: 7.7159x; 7.7159x over previous
import functools

import jax
import jax.numpy as jnp
from jax import lax
from jax.experimental import pallas as pl
from jax.experimental.pallas import tpu as pltpu
from jax.experimental.pallas import tpu_sc as plsc

K = 16
EPS = 1e-5
NEG_SLOPE = 0.1

_NC = 2
_NSUB = 16
_L = 16
_NW = _NC * _NSUB


def _lrelu(x):
    return jnp.where(x > 0, x, NEG_SLOPE * x)


def _knn_body(xf_ref, xt_ref, idx_ref, *, n, tile):
    xf = xf_ref[0]
    xt = xt_ref[0]
    g = lax.dot_general(xf.astype(jnp.bfloat16), xt.astype(jnp.bfloat16),
                        (((0,), (0,)), ((), ())),
                        preferred_element_type=jnp.float32)
    inner = -2.0 * g
    xxf = jnp.transpose(jnp.sum(xf * xf, axis=0, keepdims=True))
    xxc = jnp.sum(xt * xt, axis=0, keepdims=True)
    work = (-xxf) - inner - xxc

    rowid = lax.broadcasted_iota(jnp.int32, (n, tile), 0)
    kid = lax.broadcasted_iota(jnp.int32, (K, tile), 0)
    acc = jnp.zeros((K, tile), jnp.int32)
    for k in range(K):
        m = jnp.max(work, axis=0, keepdims=True)
        cand = jnp.where(work == m, rowid, n)
        a = jnp.min(cand, axis=0, keepdims=True)
        acc = jnp.where(kid == k, a, acc)
        work = jnp.where(rowid == a, -jnp.inf, work)
    idx_ref[0] = acc


def _knn(xyz, *, tile=256):
    b, c, n = xyz.shape
    grid = (b, n // tile)
    return pl.pallas_call(
        functools.partial(_knn_body, n=n, tile=tile),
        grid=grid,
        in_specs=[
            pl.BlockSpec((1, c, n), lambda i, j: (i, 0, 0)),
            pl.BlockSpec((1, c, tile), lambda i, j: (i, 0, j)),
        ],
        out_specs=pl.BlockSpec((1, K, tile), lambda i, j: (i, 0, j)),
        out_shape=jax.ShapeDtypeStruct((b, K, n), jnp.int32),
    )(xyz, xyz)


def _build_edge_features(xyz, idx):
    b, c, n = xyz.shape
    m = b * n
    wpb = _NW // b
    ppw = n // wpb

    chans = [xyz[:, cc, :].reshape(m) for cc in range(c)]
    idx_r = idx.reshape(b * K, 1, n)

    mesh = plsc.VectorSubcoreMesh(core_axis_name="c", subcore_axis_name="s")

    @functools.partial(
        pl.kernel,
        mesh=mesh,
        out_type=jax.ShapeDtypeStruct((3 * c * K * m,), jnp.float32),
        scratch_types=[
            pltpu.VMEM((ppw,), jnp.int32),
            pltpu.VMEM((ppw,), jnp.int32),
            pltpu.VMEM((c * ppw,), jnp.float32),
            pltpu.VMEM((c * ppw,), jnp.float32),
            pltpu.VMEM((3 * c * K * ppw,), jnp.float32),
            pltpu.SemaphoreType.DMA,
        ],
    )
    def sc_build(x0_hbm, x1_hbm, x2_hbm, idx_hbm, f_hbm,
                 idxk_v, gidx_v, p_v, q_v, f_v, sem):
        xc_hbm = (x0_hbm, x1_hbm, x2_hbm)
        wid = lax.axis_index("s") * _NC + lax.axis_index("c")
        bb = wid // wpb
        base_pt = (wid % wpb) * ppw
        gbase = bb * n + base_pt

        for cc in range(c):
            pltpu.sync_copy(xc_hbm[cc].at[pl.ds(gbase, ppw)],
                            p_v.at[pl.ds(cc * ppw, ppw)])

        for k in range(K):
            pltpu.sync_copy(idx_hbm.at[bb * K + k, 0, pl.ds(base_pt, ppw)],
                            idxk_v)

            def ibody(j, _):
                off = j * _L
                gidx_v[pl.ds(off, _L)] = idxk_v[pl.ds(off, _L)] + bb * n
                return 0

            lax.fori_loop(0, ppw // _L, ibody, 0)

            cps = [pltpu.async_copy(xc_hbm[cc].at[gidx_v],
                                    q_v.at[pl.ds(cc * ppw, ppw)], sem)
                   for cc in range(c)]
            for cp in cps:
                cp.wait()

            def jbody(j, _, k=k):
                off = j * _L
                for cc in range(c):
                    p = p_v[pl.ds(cc * ppw + off, _L)]
                    q = q_v[pl.ds(cc * ppw + off, _L)]
                    f_v[pl.ds((cc * K + k) * ppw + off, _L)] = p
                    f_v[pl.ds(((c + cc) * K + k) * ppw + off, _L)] = q
                    f_v[pl.ds(((2 * c + cc) * K + k) * ppw + off, _L)] = p - q
                return 0

            lax.fori_loop(0, ppw // _L, jbody, 0)

        for cc in range(3 * c):
            cps = [pltpu.async_copy(
                       f_v.at[pl.ds((cc * K + k) * ppw, ppw)],
                       f_hbm.at[pl.ds((cc * K + k) * m + gbase, ppw)], sem)
                   for k in range(K)]
            for cp in cps:
                cp.wait()

    return sc_build(chans[0], chans[1], chans[2], idx_r).reshape(3 * c, K, m)


def _f_moments_body(f_ref, sum_ref, s_ref):
    @pl.when(pl.program_id(0) == 0)
    def _():
        sum_ref[...] = jnp.zeros_like(sum_ref)
        s_ref[...] = jnp.zeros_like(s_ref)

    d = f_ref.shape[0]
    s = jnp.zeros((d, 1), jnp.float32)
    ss = jnp.zeros((d, d), jnp.float32)
    for k in range(K):
        fk = f_ref[:, k, :]
        s = s + jnp.sum(fk, axis=1, keepdims=True)
        ss = ss + lax.dot_general(fk, fk, (((1,), (1,)), ((), ())),
                                  preferred_element_type=jnp.float32)
    sum_ref[...] += s
    s_ref[...] += ss


def _f_moments(f, *, tile=2048):
    d, _, m = f.shape
    grid = (m // tile,)
    return pl.pallas_call(
        _f_moments_body,
        grid=grid,
        in_specs=[pl.BlockSpec((d, K, tile), lambda i: (0, 0, i))],
        out_specs=[
            pl.BlockSpec((d, 1), lambda i: (0, 0)),
            pl.BlockSpec((d, d), lambda i: (0, 0)),
        ],
        out_shape=[
            jax.ShapeDtypeStruct((d, 1), jnp.float32),
            jax.ShapeDtypeStruct((d, d), jnp.float32),
        ],
    )(f)


def _x1_moments_body(f_ref, w_ref, sc_ref, sh_ref, sum_ref, s_ref):
    @pl.when(pl.program_id(0) == 0)
    def _():
        sum_ref[...] = jnp.zeros_like(sum_ref)
        s_ref[...] = jnp.zeros_like(s_ref)

    co = w_ref.shape[0]
    w = w_ref[...]
    scale = sc_ref[...]
    shift = sh_ref[...]
    s = jnp.zeros((co, 1), jnp.float32)
    ss = jnp.zeros((co, co), jnp.float32)
    for k in range(K):
        fk = f_ref[:, k, :]
        y = lax.dot_general(w, fk, (((1,), (0,)), ((), ())),
                            preferred_element_type=jnp.float32)
        x1 = _lrelu(y * scale + shift)
        s = s + jnp.sum(x1, axis=1, keepdims=True)
        ss = ss + lax.dot_general(x1, x1, (((1,), (1,)), ((), ())),
                                  preferred_element_type=jnp.float32)
    sum_ref[...] += s
    s_ref[...] += ss


def _x1_moments(f, w1, scale1, shift1, *, tile=2048):
    d, _, m = f.shape
    co = w1.shape[0]
    grid = (m // tile,)
    return pl.pallas_call(
        _x1_moments_body,
        grid=grid,
        in_specs=[
            pl.BlockSpec((d, K, tile), lambda i: (0, 0, i)),
            pl.BlockSpec((co, d), lambda i: (0, 0)),
            pl.BlockSpec((co, 1), lambda i: (0, 0)),
            pl.BlockSpec((co, 1), lambda i: (0, 0)),
        ],
        out_specs=[
            pl.BlockSpec((co, 1), lambda i: (0, 0)),
            pl.BlockSpec((co, co), lambda i: (0, 0)),
        ],
        out_shape=[
            jax.ShapeDtypeStruct((co, 1), jnp.float32),
            jax.ShapeDtypeStruct((co, co), jnp.float32),
        ],
    )(f, w1, scale1, shift1)


def _final_body(f_ref, w1_ref, sc1_ref, sh1_ref, w2_ref, sc2_ref, sh2_ref,
                out_ref):
    w1 = w1_ref[...]
    sc1 = sc1_ref[...]
    sh1 = sh1_ref[...]
    w2 = w2_ref[...]
    sc2 = sc2_ref[...]
    sh2 = sh2_ref[...]
    co = w2.shape[0]
    tile = f_ref.shape[2]
    acc = jnp.full((co, tile), -jnp.inf, jnp.float32)
    for k in range(K):
        fk = f_ref[:, k, :]
        y1 = lax.dot_general(w1, fk, (((1,), (0,)), ((), ())),
                             preferred_element_type=jnp.float32)
        x1 = _lrelu(y1 * sc1 + sh1)
        y2 = lax.dot_general(w2, x1, (((1,), (0,)), ((), ())),
                             preferred_element_type=jnp.float32)
        x2 = _lrelu(y2 * sc2 + sh2)
        acc = jnp.maximum(acc, x2)
    out_ref[0] = acc


def _final(f, w1, scale1, shift1, w2, scale2, shift2, *, b, n, tile=512):
    d, _, m = f.shape
    co = w2.shape[0]
    nt = n // tile
    grid = (b, nt)
    return pl.pallas_call(
        _final_body,
        grid=grid,
        in_specs=[
            pl.BlockSpec((d, K, tile), lambda i, j: (0, 0, i * nt + j)),
            pl.BlockSpec((co, d), lambda i, j: (0, 0)),
            pl.BlockSpec((co, 1), lambda i, j: (0, 0)),
            pl.BlockSpec((co, 1), lambda i, j: (0, 0)),
            pl.BlockSpec((co, co), lambda i, j: (0, 0)),
            pl.BlockSpec((co, 1), lambda i, j: (0, 0)),
            pl.BlockSpec((co, 1), lambda i, j: (0, 0)),
        ],
        out_specs=pl.BlockSpec((1, co, tile), lambda i, j: (i, 0, j)),
        out_shape=jax.ShapeDtypeStruct((b, co, n), jnp.float32),
    )(f, w1, scale1, shift1, w2, scale2, shift2)


def kernel(xyz, W1, g1, b1, W2, g2, b2):
    b, c, n = xyz.shape
    e = b * n * K

    idx = _knn(xyz)
    f = _build_edge_features(xyz, idx)

    sum_f, s_f = _f_moments(f)
    mu_f = sum_f[:, 0] / e
    m2_f = s_f / e
    mean1 = W1 @ mu_f
    ey2 = jnp.sum((W1 @ m2_f) * W1, axis=1)
    var1 = ey2 - mean1 ** 2
    scale1 = g1 / jnp.sqrt(var1 + EPS)
    shift1 = b1 - mean1 * scale1

    sum_x, s_x = _x1_moments(f, W1, scale1[:, None], shift1[:, None])
    mu_x = sum_x[:, 0] / e
    m2_x = s_x / e
    mean2 = W2 @ mu_x
    ey2b = jnp.sum((W2 @ m2_x) * W2, axis=1)
    var2 = ey2b - mean2 ** 2
    scale2 = g2 / jnp.sqrt(var2 + EPS)
    shift2 = b2 - mean2 * scale2

    return _final(f, W1, scale1[:, None], shift1[:, None],
                  W2, scale2[:, None], shift2[:, None], b=b, n=n)


# speedup vs baseline: 10.1529x; 1.3158x over previous
import functools

import jax
import jax.numpy as jnp
from jax import lax
from jax.experimental import pallas as pl
from jax.experimental.pallas import tpu as pltpu
from jax.experimental.pallas import tpu_sc as plsc

K = 16
EPS = 1e-5
NEG_SLOPE = 0.1

_NC = 2
_NSUB = 16
_L = 16
_NW = _NC * _NSUB


def _lrelu(x):
    return jnp.where(x > 0, x, NEG_SLOPE * x)


def _knn_body(xf_ref, xt_ref, idx_ref, *, n, tile):
    xf = xf_ref[0]
    xt = xt_ref[0]
    g = lax.dot_general(xf.astype(jnp.bfloat16), xt.astype(jnp.bfloat16),
                        (((0,), (0,)), ((), ())),
                        preferred_element_type=jnp.float32)
    inner = -2.0 * g
    xxf = jnp.transpose(jnp.sum(xf * xf, axis=0, keepdims=True))
    xxc = jnp.sum(xt * xt, axis=0, keepdims=True)
    work = (-xxf) - inner - xxc

    rowid = lax.broadcasted_iota(jnp.int32, (n, tile), 0)
    kid = lax.broadcasted_iota(jnp.int32, (K, tile), 0)
    acc = jnp.zeros((K, tile), jnp.int32)
    for k in range(K):
        a = jnp.argmax(work, axis=0, keepdims=True)
        acc = jnp.where(kid == k, a.astype(jnp.int32), acc)
        if k < K - 1:
            work = jnp.where(rowid == a, -jnp.inf, work)
    idx_ref[0] = acc


def _knn(xyz, *, tile=256):
    b, c, n = xyz.shape
    grid = (b, n // tile)
    return pl.pallas_call(
        functools.partial(_knn_body, n=n, tile=tile),
        grid=grid,
        in_specs=[
            pl.BlockSpec((1, c, n), lambda i, j: (i, 0, 0)),
            pl.BlockSpec((1, c, tile), lambda i, j: (i, 0, j)),
        ],
        out_specs=pl.BlockSpec((1, K, tile), lambda i, j: (i, 0, j)),
        out_shape=jax.ShapeDtypeStruct((b, K, n), jnp.int32),
    )(xyz, xyz)


def _build_edge_features(xyz, idx):
    b, c, n = xyz.shape
    m = b * n
    wpb = _NW // b
    ppw = n // wpb

    chans = [xyz[:, cc, :].reshape(m) for cc in range(c)]
    idx_r = idx.reshape(b * K, 1, n)

    mesh = plsc.VectorSubcoreMesh(core_axis_name="c", subcore_axis_name="s")

    @functools.partial(
        pl.kernel,
        mesh=mesh,
        out_type=jax.ShapeDtypeStruct((3 * c * K * m,), jnp.float32),
        scratch_types=[
            pltpu.VMEM((ppw,), jnp.int32),
            pltpu.VMEM((ppw,), jnp.int32),
            pltpu.VMEM((c * ppw,), jnp.float32),
            pltpu.VMEM((c * ppw,), jnp.float32),
            pltpu.VMEM((3 * c * K * ppw,), jnp.float32),
            pltpu.SemaphoreType.DMA,
        ],
    )
    def sc_build(x0_hbm, x1_hbm, x2_hbm, idx_hbm, f_hbm,
                 idxk_v, gidx_v, p_v, q_v, f_v, sem):
        xc_hbm = (x0_hbm, x1_hbm, x2_hbm)
        wid = lax.axis_index("s") * _NC + lax.axis_index("c")
        bb = wid // wpb
        base_pt = (wid % wpb) * ppw
        gbase = bb * n + base_pt

        for cc in range(c):
            pltpu.sync_copy(xc_hbm[cc].at[pl.ds(gbase, ppw)],
                            p_v.at[pl.ds(cc * ppw, ppw)])

        for k in range(K):
            pltpu.sync_copy(idx_hbm.at[bb * K + k, 0, pl.ds(base_pt, ppw)],
                            idxk_v)

            def ibody(j, _):
                off = j * _L
                gidx_v[pl.ds(off, _L)] = idxk_v[pl.ds(off, _L)] + bb * n
                return 0

            lax.fori_loop(0, ppw // _L, ibody, 0)

            cps = [pltpu.async_copy(xc_hbm[cc].at[gidx_v],
                                    q_v.at[pl.ds(cc * ppw, ppw)], sem)
                   for cc in range(c)]
            for cp in cps:
                cp.wait()

            def jbody(j, _, k=k):
                off = j * _L
                for cc in range(c):
                    p = p_v[pl.ds(cc * ppw + off, _L)]
                    q = q_v[pl.ds(cc * ppw + off, _L)]
                    f_v[pl.ds((cc * K + k) * ppw + off, _L)] = p
                    f_v[pl.ds(((c + cc) * K + k) * ppw + off, _L)] = q
                    f_v[pl.ds(((2 * c + cc) * K + k) * ppw + off, _L)] = p - q
                return 0

            lax.fori_loop(0, ppw // _L, jbody, 0)

        for cc in range(3 * c):
            cps = [pltpu.async_copy(
                       f_v.at[pl.ds((cc * K + k) * ppw, ppw)],
                       f_hbm.at[pl.ds((cc * K + k) * m + gbase, ppw)], sem)
                   for k in range(K)]
            for cp in cps:
                cp.wait()

    return sc_build(chans[0], chans[1], chans[2], idx_r).reshape(3 * c, K, m)


def _f_moments_body(f_ref, sum_ref, s_ref):
    @pl.when(pl.program_id(0) == 0)
    def _():
        sum_ref[...] = jnp.zeros_like(sum_ref)
        s_ref[...] = jnp.zeros_like(s_ref)

    d = f_ref.shape[0]
    s = jnp.zeros((d, 1), jnp.float32)
    ss = jnp.zeros((d, d), jnp.float32)
    for k in range(K):
        fk = f_ref[:, k, :]
        s = s + jnp.sum(fk, axis=1, keepdims=True)
        ss = ss + lax.dot_general(fk, fk, (((1,), (1,)), ((), ())),
                                  preferred_element_type=jnp.float32)
    sum_ref[...] += s
    s_ref[...] += ss


def _f_moments(f, *, tile=2048):
    d, _, m = f.shape
    grid = (m // tile,)
    return pl.pallas_call(
        _f_moments_body,
        grid=grid,
        in_specs=[pl.BlockSpec((d, K, tile), lambda i: (0, 0, i))],
        out_specs=[
            pl.BlockSpec((d, 1), lambda i: (0, 0)),
            pl.BlockSpec((d, d), lambda i: (0, 0)),
        ],
        out_shape=[
            jax.ShapeDtypeStruct((d, 1), jnp.float32),
            jax.ShapeDtypeStruct((d, d), jnp.float32),
        ],
    )(f)


def _x1_moments_body(f_ref, w_ref, sc_ref, sh_ref, sum_ref, s_ref):
    @pl.when(pl.program_id(0) == 0)
    def _():
        sum_ref[...] = jnp.zeros_like(sum_ref)
        s_ref[...] = jnp.zeros_like(s_ref)

    co = w_ref.shape[0]
    w = w_ref[...]
    scale = sc_ref[...]
    shift = sh_ref[...]
    s = jnp.zeros((co, 1), jnp.float32)
    ss = jnp.zeros((co, co), jnp.float32)
    for k in range(K):
        fk = f_ref[:, k, :]
        y = lax.dot_general(w, fk, (((1,), (0,)), ((), ())),
                            preferred_element_type=jnp.float32)
        x1 = _lrelu(y * scale + shift)
        s = s + jnp.sum(x1, axis=1, keepdims=True)
        ss = ss + lax.dot_general(x1, x1, (((1,), (1,)), ((), ())),
                                  preferred_element_type=jnp.float32)
    sum_ref[...] += s
    s_ref[...] += ss


def _x1_moments(f, w1, scale1, shift1, *, tile=2048):
    d, _, m = f.shape
    co = w1.shape[0]
    grid = (m // tile,)
    return pl.pallas_call(
        _x1_moments_body,
        grid=grid,
        in_specs=[
            pl.BlockSpec((d, K, tile), lambda i: (0, 0, i)),
            pl.BlockSpec((co, d), lambda i: (0, 0)),
            pl.BlockSpec((co, 1), lambda i: (0, 0)),
            pl.BlockSpec((co, 1), lambda i: (0, 0)),
        ],
        out_specs=[
            pl.BlockSpec((co, 1), lambda i: (0, 0)),
            pl.BlockSpec((co, co), lambda i: (0, 0)),
        ],
        out_shape=[
            jax.ShapeDtypeStruct((co, 1), jnp.float32),
            jax.ShapeDtypeStruct((co, co), jnp.float32),
        ],
    )(f, w1, scale1, shift1)


def _final_body(f_ref, w1_ref, sc1_ref, sh1_ref, w2_ref, sc2_ref, sh2_ref,
                out_ref):
    w1 = w1_ref[...]
    sc1 = sc1_ref[...]
    sh1 = sh1_ref[...]
    w2 = w2_ref[...]
    sc2 = sc2_ref[...]
    sh2 = sh2_ref[...]
    co = w2.shape[0]
    tile = f_ref.shape[2]
    acc = jnp.full((co, tile), -jnp.inf, jnp.float32)
    for k in range(K):
        fk = f_ref[:, k, :]
        y1 = lax.dot_general(w1, fk, (((1,), (0,)), ((), ())),
                             preferred_element_type=jnp.float32)
        x1 = _lrelu(y1 * sc1 + sh1)
        y2 = lax.dot_general(w2, x1, (((1,), (0,)), ((), ())),
                             preferred_element_type=jnp.float32)
        x2 = _lrelu(y2 * sc2 + sh2)
        acc = jnp.maximum(acc, x2)
    out_ref[0] = acc


def _final(f, w1, scale1, shift1, w2, scale2, shift2, *, b, n, tile=512):
    d, _, m = f.shape
    co = w2.shape[0]
    nt = n // tile
    grid = (b, nt)
    return pl.pallas_call(
        _final_body,
        grid=grid,
        in_specs=[
            pl.BlockSpec((d, K, tile), lambda i, j: (0, 0, i * nt + j)),
            pl.BlockSpec((co, d), lambda i, j: (0, 0)),
            pl.BlockSpec((co, 1), lambda i, j: (0, 0)),
            pl.BlockSpec((co, 1), lambda i, j: (0, 0)),
            pl.BlockSpec((co, co), lambda i, j: (0, 0)),
            pl.BlockSpec((co, 1), lambda i, j: (0, 0)),
            pl.BlockSpec((co, 1), lambda i, j: (0, 0)),
        ],
        out_specs=pl.BlockSpec((1, co, tile), lambda i, j: (i, 0, j)),
        out_shape=jax.ShapeDtypeStruct((b, co, n), jnp.float32),
    )(f, w1, scale1, shift1, w2, scale2, shift2)


def kernel(xyz, W1, g1, b1, W2, g2, b2):
    b, c, n = xyz.shape
    e = b * n * K

    idx = _knn(xyz)
    f = _build_edge_features(xyz, idx)

    sum_f, s_f = _f_moments(f)
    mu_f = sum_f[:, 0] / e
    m2_f = s_f / e
    mean1 = W1 @ mu_f
    ey2 = jnp.sum((W1 @ m2_f) * W1, axis=1)
    var1 = ey2 - mean1 ** 2
    scale1 = g1 / jnp.sqrt(var1 + EPS)
    shift1 = b1 - mean1 * scale1

    sum_x, s_x = _x1_moments(f, W1, scale1[:, None], shift1[:, None])
    mu_x = sum_x[:, 0] / e
    m2_x = s_x / e
    mean2 = W2 @ mu_x
    ey2b = jnp.sum((W2 @ m2_x) * W2, axis=1)
    var2 = ey2b - mean2 ** 2
    scale2 = g2 / jnp.sqrt(var2 + EPS)
    shift2 = b2 - mean2 * scale2

    return _final(f, W1, scale1[:, None], shift1[:, None],
                  W2, scale2[:, None], shift2[:, None], b=b, n=n)


# speedup vs baseline: 10.2496x; 1.0095x over previous
import functools

import jax
import jax.numpy as jnp
from jax import lax
from jax.experimental import pallas as pl
from jax.experimental.pallas import tpu as pltpu
from jax.experimental.pallas import tpu_sc as plsc

K = 16
EPS = 1e-5
NEG_SLOPE = 0.1

_NC = 2
_NSUB = 16
_L = 16
_NW = _NC * _NSUB


def _lrelu(x):
    return jnp.where(x > 0, x, NEG_SLOPE * x)


def _knn_body(xf_ref, xt_ref, idx_ref, *, n, tile):
    xf = xf_ref[0]
    xt = xt_ref[0]
    g = lax.dot_general(xf.astype(jnp.bfloat16), xt.astype(jnp.bfloat16),
                        (((0,), (0,)), ((), ())),
                        preferred_element_type=jnp.float32)
    inner = -2.0 * g
    xxf = jnp.transpose(jnp.sum(xf * xf, axis=0, keepdims=True))
    xxc = jnp.sum(xt * xt, axis=0, keepdims=True)
    work = (-xxf) - inner - xxc

    rowid = lax.broadcasted_iota(jnp.int32, (n, tile), 0)
    kid = lax.broadcasted_iota(jnp.int32, (K, tile), 0)
    acc = jnp.zeros((K, tile), jnp.int32)
    for k in range(K):
        a = jnp.argmax(work, axis=0, keepdims=True)
        acc = jnp.where(kid == k, a.astype(jnp.int32), acc)
        if k < K - 1:
            work = jnp.where(rowid == a, -jnp.inf, work)
    idx_ref[0] = acc


def _knn(xyz, *, tile=256):
    b, c, n = xyz.shape
    grid = (b, n // tile)
    return pl.pallas_call(
        functools.partial(_knn_body, n=n, tile=tile),
        grid=grid,
        in_specs=[
            pl.BlockSpec((1, c, n), lambda i, j: (i, 0, 0)),
            pl.BlockSpec((1, c, tile), lambda i, j: (i, 0, j)),
        ],
        out_specs=pl.BlockSpec((1, K, tile), lambda i, j: (i, 0, j)),
        out_shape=jax.ShapeDtypeStruct((b, K, n), jnp.int32),
    )(xyz, xyz)


def _build_edge_features(xyz, idx):
    b, c, n = xyz.shape
    m = b * n
    wpb = _NW // b
    ppw = n // wpb

    xyz_flat = xyz.reshape(b * c * n)
    idx_r = idx.reshape(b * K, 1, n)

    mesh = plsc.VectorSubcoreMesh(core_axis_name="c", subcore_axis_name="s")

    @functools.partial(
        pl.kernel,
        mesh=mesh,
        out_type=jax.ShapeDtypeStruct((3 * c * K * m,), jnp.float32),
        scratch_types=[
            pltpu.VMEM((ppw,), jnp.int32),
            pltpu.VMEM((ppw,), jnp.int32),
            pltpu.VMEM((ppw,), jnp.int32),
            pltpu.VMEM((ppw,), jnp.int32),
            pltpu.VMEM((c * ppw,), jnp.float32),
            pltpu.VMEM((c * ppw,), jnp.float32),
            pltpu.VMEM((3 * c * K * ppw,), jnp.float32),
            pltpu.SemaphoreType.DMA,
        ],
    )
    def sc_build(x_hbm, idx_hbm, f_hbm, idxk_v, g0_v, g1_v, g2_v,
                 p_v, q_v, f_v, sem):
        gidx = (g0_v, g1_v, g2_v)
        wid = lax.axis_index("s") * _NC + lax.axis_index("c")
        bb = wid // wpb
        base_pt = (wid % wpb) * ppw
        gbase = bb * n + base_pt

        for cc in range(c):
            pltpu.sync_copy(x_hbm.at[pl.ds((bb * c + cc) * n + base_pt, ppw)],
                            p_v.at[pl.ds(cc * ppw, ppw)])

        for k in range(K):
            pltpu.sync_copy(idx_hbm.at[bb * K + k, 0, pl.ds(base_pt, ppw)],
                            idxk_v)

            def ibody(j, _):
                off = j * _L
                nbr = idxk_v[pl.ds(off, _L)]
                for cc in range(c):
                    gidx[cc][pl.ds(off, _L)] = nbr + (bb * c + cc) * n
                return 0

            lax.fori_loop(0, ppw // _L, ibody, 0)

            cps = [pltpu.async_copy(x_hbm.at[gidx[cc]],
                                    q_v.at[pl.ds(cc * ppw, ppw)], sem)
                   for cc in range(c)]
            for cp in cps:
                cp.wait()

            def jbody(j, _, k=k):
                off = j * _L
                for cc in range(c):
                    p = p_v[pl.ds(cc * ppw + off, _L)]
                    q = q_v[pl.ds(cc * ppw + off, _L)]
                    f_v[pl.ds((cc * K + k) * ppw + off, _L)] = p
                    f_v[pl.ds(((c + cc) * K + k) * ppw + off, _L)] = q
                    f_v[pl.ds(((2 * c + cc) * K + k) * ppw + off, _L)] = p - q
                return 0

            lax.fori_loop(0, ppw // _L, jbody, 0)

        for cc in range(3 * c):
            cps = [pltpu.async_copy(
                       f_v.at[pl.ds((cc * K + k) * ppw, ppw)],
                       f_hbm.at[pl.ds((cc * K + k) * m + gbase, ppw)], sem)
                   for k in range(K)]
            for cp in cps:
                cp.wait()

    return sc_build(xyz_flat, idx_r).reshape(3 * c, K, m)


def _f_moments_body(f_ref, sum_ref, s_ref):
    @pl.when(pl.program_id(0) == 0)
    def _():
        sum_ref[...] = jnp.zeros_like(sum_ref)
        s_ref[...] = jnp.zeros_like(s_ref)

    d = f_ref.shape[0]
    s = jnp.zeros((d, 1), jnp.float32)
    ss = jnp.zeros((d, d), jnp.float32)
    for k in range(K):
        fk = f_ref[:, k, :]
        s = s + jnp.sum(fk, axis=1, keepdims=True)
        ss = ss + lax.dot_general(fk, fk, (((1,), (1,)), ((), ())),
                                  preferred_element_type=jnp.float32)
    sum_ref[...] += s
    s_ref[...] += ss


def _f_moments(f, *, tile=2048):
    d, _, m = f.shape
    grid = (m // tile,)
    return pl.pallas_call(
        _f_moments_body,
        grid=grid,
        in_specs=[pl.BlockSpec((d, K, tile), lambda i: (0, 0, i))],
        out_specs=[
            pl.BlockSpec((d, 1), lambda i: (0, 0)),
            pl.BlockSpec((d, d), lambda i: (0, 0)),
        ],
        out_shape=[
            jax.ShapeDtypeStruct((d, 1), jnp.float32),
            jax.ShapeDtypeStruct((d, d), jnp.float32),
        ],
    )(f)


def _bn_scale_shift(w, sum_ref, s_ref, g_ref, b_ref, e):
    mu = sum_ref[...] / e
    m2 = s_ref[...] / e
    hp = jax.lax.Precision.HIGHEST
    mean = lax.dot_general(w, mu, (((1,), (0,)), ((), ())),
                           preferred_element_type=jnp.float32, precision=hp)
    wm = lax.dot_general(w, m2, (((1,), (0,)), ((), ())),
                         preferred_element_type=jnp.float32, precision=hp)
    ey2 = jnp.sum(wm * w, axis=1, keepdims=True)
    var = ey2 - mean * mean
    scale = g_ref[...] / jnp.sqrt(var + EPS)
    shift = b_ref[...] - mean * scale
    return scale, shift


def _x1_moments_body(f_ref, w_ref, sumf_ref, sf_ref, g_ref, b_ref,
                     sum_ref, s_ref, sc_v, sh_v, *, e):
    @pl.when(pl.program_id(0) == 0)
    def _():
        sum_ref[...] = jnp.zeros_like(sum_ref)
        s_ref[...] = jnp.zeros_like(s_ref)
        scale, shift = _bn_scale_shift(w_ref[...], sumf_ref, sf_ref,
                                       g_ref, b_ref, e)
        sc_v[...] = scale
        sh_v[...] = shift

    co = w_ref.shape[0]
    w = w_ref[...]
    scale = sc_v[...]
    shift = sh_v[...]
    s = jnp.zeros((co, 1), jnp.float32)
    ss = jnp.zeros((co, co), jnp.float32)
    for k in range(K):
        fk = f_ref[:, k, :]
        y = lax.dot_general(w, fk, (((1,), (0,)), ((), ())),
                            preferred_element_type=jnp.float32)
        x1 = _lrelu(y * scale + shift)
        s = s + jnp.sum(x1, axis=1, keepdims=True)
        ss = ss + lax.dot_general(x1, x1, (((1,), (1,)), ((), ())),
                                  preferred_element_type=jnp.float32)
    sum_ref[...] += s
    s_ref[...] += ss


def _x1_moments(f, w1, sum_f, s_f, g1, b1, *, e, tile=2048):
    d, _, m = f.shape
    co = w1.shape[0]
    grid = (m // tile,)
    return pl.pallas_call(
        functools.partial(_x1_moments_body, e=e),
        grid=grid,
        in_specs=[
            pl.BlockSpec((d, K, tile), lambda i: (0, 0, i)),
            pl.BlockSpec((co, d), lambda i: (0, 0)),
            pl.BlockSpec((d, 1), lambda i: (0, 0)),
            pl.BlockSpec((d, d), lambda i: (0, 0)),
            pl.BlockSpec((co, 1), lambda i: (0, 0)),
            pl.BlockSpec((co, 1), lambda i: (0, 0)),
        ],
        out_specs=[
            pl.BlockSpec((co, 1), lambda i: (0, 0)),
            pl.BlockSpec((co, co), lambda i: (0, 0)),
        ],
        out_shape=[
            jax.ShapeDtypeStruct((co, 1), jnp.float32),
            jax.ShapeDtypeStruct((co, co), jnp.float32),
        ],
        scratch_shapes=[
            pltpu.VMEM((co, 1), jnp.float32),
            pltpu.VMEM((co, 1), jnp.float32),
        ],
    )(f, w1, sum_f, s_f, g1, b1)


def _final_body(f_ref, w1_ref, sumf_ref, sf_ref, g1_ref, b1_ref,
                w2_ref, sumx_ref, sx_ref, g2_ref, b2_ref, out_ref,
                sc1_v, sh1_v, sc2_v, sh2_v, *, e):
    @pl.when(jnp.logical_and(pl.program_id(0) == 0, pl.program_id(1) == 0))
    def _():
        s1, h1 = _bn_scale_shift(w1_ref[...], sumf_ref, sf_ref,
                                 g1_ref, b1_ref, e)
        sc1_v[...] = s1
        sh1_v[...] = h1
        s2, h2 = _bn_scale_shift(w2_ref[...], sumx_ref, sx_ref,
                                 g2_ref, b2_ref, e)
        sc2_v[...] = s2
        sh2_v[...] = h2

    w1 = w1_ref[...]
    sc1 = sc1_v[...]
    sh1 = sh1_v[...]
    w2 = w2_ref[...]
    sc2 = sc2_v[...]
    sh2 = sh2_v[...]
    co = w2.shape[0]
    tile = f_ref.shape[2]
    acc = jnp.full((co, tile), -jnp.inf, jnp.float32)
    for k in range(K):
        fk = f_ref[:, k, :]
        y1 = lax.dot_general(w1, fk, (((1,), (0,)), ((), ())),
                             preferred_element_type=jnp.float32)
        x1 = _lrelu(y1 * sc1 + sh1)
        y2 = lax.dot_general(w2, x1, (((1,), (0,)), ((), ())),
                             preferred_element_type=jnp.float32)
        x2 = _lrelu(y2 * sc2 + sh2)
        acc = jnp.maximum(acc, x2)
    out_ref[0] = acc


def _final(f, w1, sum_f, s_f, g1, b1, w2, sum_x, s_x, g2, b2,
           *, e, b, n, tile=512):
    d, _, m = f.shape
    co = w2.shape[0]
    nt = n // tile
    grid = (b, nt)
    return pl.pallas_call(
        functools.partial(_final_body, e=e),
        grid=grid,
        in_specs=[
            pl.BlockSpec((d, K, tile), lambda i, j: (0, 0, i * nt + j)),
            pl.BlockSpec((co, d), lambda i, j: (0, 0)),
            pl.BlockSpec((d, 1), lambda i, j: (0, 0)),
            pl.BlockSpec((d, d), lambda i, j: (0, 0)),
            pl.BlockSpec((co, 1), lambda i, j: (0, 0)),
            pl.BlockSpec((co, 1), lambda i, j: (0, 0)),
            pl.BlockSpec((co, co), lambda i, j: (0, 0)),
            pl.BlockSpec((co, 1), lambda i, j: (0, 0)),
            pl.BlockSpec((co, co), lambda i, j: (0, 0)),
            pl.BlockSpec((co, 1), lambda i, j: (0, 0)),
            pl.BlockSpec((co, 1), lambda i, j: (0, 0)),
        ],
        out_specs=pl.BlockSpec((1, co, tile), lambda i, j: (i, 0, j)),
        out_shape=jax.ShapeDtypeStruct((b, co, n), jnp.float32),
        scratch_shapes=[
            pltpu.VMEM((co, 1), jnp.float32),
            pltpu.VMEM((co, 1), jnp.float32),
            pltpu.VMEM((co, 1), jnp.float32),
            pltpu.VMEM((co, 1), jnp.float32),
        ],
    )(f, w1, sum_f, s_f, g1, b1, w2, sum_x, s_x, g2, b2)


def kernel(xyz, W1, g1, b1, W2, g2, b2):
    b, c, n = xyz.shape
    e = b * n * K

    idx = _knn(xyz)
    f = _build_edge_features(xyz, idx)
    sum_f, s_f = _f_moments(f)
    g1c, b1c = g1[:, None], b1[:, None]
    g2c, b2c = g2[:, None], b2[:, None]
    sum_x, s_x = _x1_moments(f, W1, sum_f, s_f, g1c, b1c, e=e)
    return _final(f, W1, sum_f, s_f, g1c, b1c,
                  W2, sum_x, s_x, g2c, b2c, e=e, b=b, n=n)


# speedup vs baseline: 11.3235x; 1.1048x over previous
import functools

import jax
import jax.numpy as jnp
from jax import lax
from jax.experimental import pallas as pl
from jax.experimental.pallas import tpu as pltpu
from jax.experimental.pallas import tpu_sc as plsc

K = 16
EPS = 1e-5
NEG_SLOPE = 0.1

_NC = 2
_NSUB = 16
_L = 16
_NW = _NC * _NSUB


def _lrelu(x):
    return jnp.where(x > 0, x, NEG_SLOPE * x)


def _knn_body(xf_ref, xt_ref, idx_ref, *, n, tile):
    xf = xf_ref[0]
    xt = xt_ref[0]
    g = lax.dot_general(xf.astype(jnp.bfloat16), xt.astype(jnp.bfloat16),
                        (((0,), (0,)), ((), ())),
                        preferred_element_type=jnp.float32)
    inner = -2.0 * g
    xxf = jnp.transpose(jnp.sum(xf * xf, axis=0, keepdims=True))
    xxc = jnp.sum(xt * xt, axis=0, keepdims=True)
    work = (-xxf) - inner - xxc

    rowid = lax.broadcasted_iota(jnp.int32, (n, tile), 0)
    kid = lax.broadcasted_iota(jnp.int32, (K, tile), 0)
    acc = jnp.zeros((K, tile), jnp.int32)
    for k in range(K):
        a = jnp.argmax(work, axis=0, keepdims=True)
        acc = jnp.where(kid == k, a.astype(jnp.int32), acc)
        if k < K - 1:
            work = jnp.where(rowid == a, -jnp.inf, work)
    idx_ref[0] = acc


def _knn(xyz, *, tile=256):
    b, c, n = xyz.shape
    grid = (b, n // tile)
    return pl.pallas_call(
        functools.partial(_knn_body, n=n, tile=tile),
        grid=grid,
        in_specs=[
            pl.BlockSpec((1, c, n), lambda i, j: (i, 0, 0)),
            pl.BlockSpec((1, c, tile), lambda i, j: (i, 0, j)),
        ],
        out_specs=pl.BlockSpec((1, K, tile), lambda i, j: (i, 0, j)),
        out_shape=jax.ShapeDtypeStruct((b, K, n), jnp.int32),
    )(xyz, xyz)


def _build_edge_features(xyz, idx):
    b, c, n = xyz.shape
    m = b * n
    wpb = _NW // b
    ppw = n // wpb

    xyz_flat = xyz.reshape(b * c * n)
    idx_r = idx.reshape(b * K, 1, n)

    mesh = plsc.VectorSubcoreMesh(core_axis_name="c", subcore_axis_name="s")

    @functools.partial(
        pl.kernel,
        mesh=mesh,
        out_type=jax.ShapeDtypeStruct((3 * c * K * m,), jnp.float32),
        scratch_types=[
            pltpu.VMEM((ppw,), jnp.int32),
            pltpu.VMEM((ppw,), jnp.int32),
            pltpu.VMEM((ppw,), jnp.int32),
            pltpu.VMEM((ppw,), jnp.int32),
            pltpu.VMEM((c * ppw,), jnp.float32),
            pltpu.VMEM((c * ppw,), jnp.float32),
            pltpu.VMEM((3 * c * K * ppw,), jnp.float32),
            pltpu.SemaphoreType.DMA,
        ],
    )
    def sc_build(x_hbm, idx_hbm, f_hbm, idxk_v, g0_v, g1_v, g2_v,
                 p_v, q_v, f_v, sem):
        gidx = (g0_v, g1_v, g2_v)
        wid = lax.axis_index("s") * _NC + lax.axis_index("c")
        bb = wid // wpb
        base_pt = (wid % wpb) * ppw
        gbase = bb * n + base_pt

        for cc in range(c):
            pltpu.sync_copy(x_hbm.at[pl.ds((bb * c + cc) * n + base_pt, ppw)],
                            p_v.at[pl.ds(cc * ppw, ppw)])

        for k in range(K):
            pltpu.sync_copy(idx_hbm.at[bb * K + k, 0, pl.ds(base_pt, ppw)],
                            idxk_v)

            def ibody(j, _):
                off = j * _L
                nbr = idxk_v[pl.ds(off, _L)]
                for cc in range(c):
                    gidx[cc][pl.ds(off, _L)] = nbr + (bb * c + cc) * n
                return 0

            lax.fori_loop(0, ppw // _L, ibody, 0)

            cps = [pltpu.async_copy(x_hbm.at[gidx[cc]],
                                    q_v.at[pl.ds(cc * ppw, ppw)], sem)
                   for cc in range(c)]
            for cp in cps:
                cp.wait()

            def jbody(j, _, k=k):
                off = j * _L
                for cc in range(c):
                    p = p_v[pl.ds(cc * ppw + off, _L)]
                    q = q_v[pl.ds(cc * ppw + off, _L)]
                    f_v[pl.ds((cc * K + k) * ppw + off, _L)] = p
                    f_v[pl.ds(((c + cc) * K + k) * ppw + off, _L)] = q
                    f_v[pl.ds(((2 * c + cc) * K + k) * ppw + off, _L)] = p - q
                return 0

            lax.fori_loop(0, ppw // _L, jbody, 0)

        for cc in range(3 * c):
            cps = [pltpu.async_copy(
                       f_v.at[pl.ds((cc * K + k) * ppw, ppw)],
                       f_hbm.at[pl.ds((cc * K + k) * m + gbase, ppw)], sem)
                   for k in range(K)]
            for cp in cps:
                cp.wait()

    return sc_build(xyz_flat, idx_r).reshape(3 * c, K, m)


def _f_moments_body(f_ref, sum_ref, s_ref):
    @pl.when(pl.program_id(0) == 0)
    def _():
        sum_ref[...] = jnp.zeros_like(sum_ref)
        s_ref[...] = jnp.zeros_like(s_ref)

    d = f_ref.shape[0]
    s = jnp.zeros((d, 1), jnp.float32)
    ss = jnp.zeros((d, d), jnp.float32)
    for k in range(K):
        fk = f_ref[:, k, :]
        s = s + jnp.sum(fk, axis=1, keepdims=True)
        ss = ss + lax.dot_general(fk, fk, (((1,), (1,)), ((), ())),
                                  preferred_element_type=jnp.float32)
    sum_ref[...] += s
    s_ref[...] += ss


def _f_moments(f, *, tile=2048):
    d, _, m = f.shape
    grid = (m // tile,)
    return pl.pallas_call(
        _f_moments_body,
        grid=grid,
        in_specs=[pl.BlockSpec((d, K, tile), lambda i: (0, 0, i))],
        out_specs=[
            pl.BlockSpec((d, 1), lambda i: (0, 0)),
            pl.BlockSpec((d, d), lambda i: (0, 0)),
        ],
        out_shape=[
            jax.ShapeDtypeStruct((d, 1), jnp.float32),
            jax.ShapeDtypeStruct((d, d), jnp.float32),
        ],
    )(f)


def _bn_scale_shift(w, sum_ref, s_ref, g_ref, b_ref, e):
    mu = sum_ref[...] / e
    m2 = s_ref[...] / e
    hp = jax.lax.Precision.HIGHEST
    mean = lax.dot_general(w, mu, (((1,), (0,)), ((), ())),
                           preferred_element_type=jnp.float32, precision=hp)
    wm = lax.dot_general(w, m2, (((1,), (0,)), ((), ())),
                         preferred_element_type=jnp.float32, precision=hp)
    ey2 = jnp.sum(wm * w, axis=1, keepdims=True)
    var = ey2 - mean * mean
    scale = g_ref[...] / jnp.sqrt(var + EPS)
    shift = b_ref[...] - mean * scale
    return scale, shift


def _x1_moments_body(f_ref, w_ref, sumf_ref, sf_ref, g_ref, b_ref,
                     sum_ref, s_ref, sc_v, sh_v, *, e):
    @pl.when(pl.program_id(0) == 0)
    def _():
        sum_ref[...] = jnp.zeros_like(sum_ref)
        s_ref[...] = jnp.zeros_like(s_ref)
        scale, shift = _bn_scale_shift(w_ref[...], sumf_ref, sf_ref,
                                       g_ref, b_ref, e)
        sc_v[...] = scale
        sh_v[...] = shift

    w = w_ref[...]
    scale = sc_v[...]
    shift = sh_v[...]
    fall = jnp.concatenate([f_ref[:, k, :] for k in range(K)], axis=1)
    y = lax.dot_general(w, fall, (((1,), (0,)), ((), ())),
                        preferred_element_type=jnp.float32)
    x1 = _lrelu(y * scale + shift)
    sum_ref[...] += jnp.sum(x1, axis=1, keepdims=True)
    s_ref[...] += lax.dot_general(x1, x1, (((1,), (1,)), ((), ())),
                                  preferred_element_type=jnp.float32)


def _x1_moments(f, w1, sum_f, s_f, g1, b1, *, e, tile=2048):
    d, _, m = f.shape
    co = w1.shape[0]
    grid = (m // tile,)
    return pl.pallas_call(
        functools.partial(_x1_moments_body, e=e),
        grid=grid,
        in_specs=[
            pl.BlockSpec((d, K, tile), lambda i: (0, 0, i)),
            pl.BlockSpec((co, d), lambda i: (0, 0)),
            pl.BlockSpec((d, 1), lambda i: (0, 0)),
            pl.BlockSpec((d, d), lambda i: (0, 0)),
            pl.BlockSpec((co, 1), lambda i: (0, 0)),
            pl.BlockSpec((co, 1), lambda i: (0, 0)),
        ],
        out_specs=[
            pl.BlockSpec((co, 1), lambda i: (0, 0)),
            pl.BlockSpec((co, co), lambda i: (0, 0)),
        ],
        out_shape=[
            jax.ShapeDtypeStruct((co, 1), jnp.float32),
            jax.ShapeDtypeStruct((co, co), jnp.float32),
        ],
        scratch_shapes=[
            pltpu.VMEM((co, 1), jnp.float32),
            pltpu.VMEM((co, 1), jnp.float32),
        ],
    )(f, w1, sum_f, s_f, g1, b1)


def _final_body(f_ref, w1_ref, sumf_ref, sf_ref, g1_ref, b1_ref,
                w2_ref, sumx_ref, sx_ref, g2_ref, b2_ref, out_ref,
                sc1_v, sh1_v, sc2_v, sh2_v, *, e):
    @pl.when(jnp.logical_and(pl.program_id(0) == 0, pl.program_id(1) == 0))
    def _():
        s1, h1 = _bn_scale_shift(w1_ref[...], sumf_ref, sf_ref,
                                 g1_ref, b1_ref, e)
        sc1_v[...] = s1
        sh1_v[...] = h1
        s2, h2 = _bn_scale_shift(w2_ref[...], sumx_ref, sx_ref,
                                 g2_ref, b2_ref, e)
        sc2_v[...] = s2
        sh2_v[...] = h2

    w1 = w1_ref[...]
    sc1 = sc1_v[...]
    sh1 = sh1_v[...]
    w2 = w2_ref[...]
    sc2 = sc2_v[...]
    sh2 = sh2_v[...]
    tile = f_ref.shape[2]
    fall = jnp.concatenate([f_ref[:, k, :] for k in range(K)], axis=1)
    y1 = lax.dot_general(w1, fall, (((1,), (0,)), ((), ())),
                         preferred_element_type=jnp.float32)
    x1 = _lrelu(y1 * sc1 + sh1)
    y2 = lax.dot_general(w2, x1, (((1,), (0,)), ((), ())),
                         preferred_element_type=jnp.float32)
    x2 = _lrelu(y2 * sc2 + sh2)
    acc = x2[:, :tile]
    for k in range(1, K):
        acc = jnp.maximum(acc, x2[:, k * tile:(k + 1) * tile])
    out_ref[0] = acc


def _final(f, w1, sum_f, s_f, g1, b1, w2, sum_x, s_x, g2, b2,
           *, e, b, n, tile=512):
    d, _, m = f.shape
    co = w2.shape[0]
    nt = n // tile
    grid = (b, nt)
    return pl.pallas_call(
        functools.partial(_final_body, e=e),
        grid=grid,
        in_specs=[
            pl.BlockSpec((d, K, tile), lambda i, j: (0, 0, i * nt + j)),
            pl.BlockSpec((co, d), lambda i, j: (0, 0)),
            pl.BlockSpec((d, 1), lambda i, j: (0, 0)),
            pl.BlockSpec((d, d), lambda i, j: (0, 0)),
            pl.BlockSpec((co, 1), lambda i, j: (0, 0)),
            pl.BlockSpec((co, 1), lambda i, j: (0, 0)),
            pl.BlockSpec((co, co), lambda i, j: (0, 0)),
            pl.BlockSpec((co, 1), lambda i, j: (0, 0)),
            pl.BlockSpec((co, co), lambda i, j: (0, 0)),
            pl.BlockSpec((co, 1), lambda i, j: (0, 0)),
            pl.BlockSpec((co, 1), lambda i, j: (0, 0)),
        ],
        out_specs=pl.BlockSpec((1, co, tile), lambda i, j: (i, 0, j)),
        out_shape=jax.ShapeDtypeStruct((b, co, n), jnp.float32),
        scratch_shapes=[
            pltpu.VMEM((co, 1), jnp.float32),
            pltpu.VMEM((co, 1), jnp.float32),
            pltpu.VMEM((co, 1), jnp.float32),
            pltpu.VMEM((co, 1), jnp.float32),
        ],
    )(f, w1, sum_f, s_f, g1, b1, w2, sum_x, s_x, g2, b2)


def kernel(xyz, W1, g1, b1, W2, g2, b2):
    b, c, n = xyz.shape
    e = b * n * K

    idx = _knn(xyz)
    f = _build_edge_features(xyz, idx)
    sum_f, s_f = _f_moments(f)
    g1c, b1c = g1[:, None], b1[:, None]
    g2c, b2c = g2[:, None], b2[:, None]
    sum_x, s_x = _x1_moments(f, W1, sum_f, s_f, g1c, b1c, e=e)
    return _final(f, W1, sum_f, s_f, g1c, b1c,
                  W2, sum_x, s_x, g2c, b2c, e=e, b=b, n=n)


# speedup vs baseline: 11.3452x; 1.0019x over previous
import functools

import jax
import jax.numpy as jnp
from jax import lax
from jax.experimental import pallas as pl
from jax.experimental.pallas import tpu as pltpu
from jax.experimental.pallas import tpu_sc as plsc

K = 16
EPS = 1e-5
NEG_SLOPE = 0.1

_NC = 2
_NSUB = 16
_L = 16
_NW = _NC * _NSUB


def _lrelu(x):
    return jnp.where(x > 0, x, NEG_SLOPE * x)


def _knn_body(xf_ref, xt_ref, idx_ref, *, n, tile):
    xf = xf_ref[0]
    xt = xt_ref[0]
    g = lax.dot_general(xf.astype(jnp.bfloat16), xt.astype(jnp.bfloat16),
                        (((0,), (0,)), ((), ())),
                        preferred_element_type=jnp.float32)
    inner = -2.0 * g
    xxf = jnp.transpose(jnp.sum(xf * xf, axis=0, keepdims=True))
    xxc = jnp.sum(xt * xt, axis=0, keepdims=True)
    work = (-xxf) - inner - xxc

    rowid = lax.broadcasted_iota(jnp.int32, (n, tile), 0)
    kid = lax.broadcasted_iota(jnp.int32, (K, tile), 0)
    acc = jnp.zeros((K, tile), jnp.int32)
    for k in range(K):
        a = jnp.argmax(work, axis=0, keepdims=True)
        acc = jnp.where(kid == k, a.astype(jnp.int32), acc)
        if k < K - 1:
            work = jnp.where(rowid == a, -jnp.inf, work)
    idx_ref[0] = acc


def _knn(xyz, *, tile=256):
    b, c, n = xyz.shape
    grid = (b, n // tile)
    return pl.pallas_call(
        functools.partial(_knn_body, n=n, tile=tile),
        grid=grid,
        in_specs=[
            pl.BlockSpec((1, c, n), lambda i, j: (i, 0, 0)),
            pl.BlockSpec((1, c, tile), lambda i, j: (i, 0, j)),
        ],
        out_specs=pl.BlockSpec((1, K, tile), lambda i, j: (i, 0, j)),
        out_shape=jax.ShapeDtypeStruct((b, K, n), jnp.int32),
    )(xyz, xyz)


def _build_edge_features(xyz, idx):
    b, c, n = xyz.shape
    m = b * n
    wpb = _NW // b
    ppw = n // wpb

    xyz_flat = xyz.reshape(b * c * n)
    idx_r = idx.reshape(b * K, 1, n)

    mesh = plsc.VectorSubcoreMesh(core_axis_name="c", subcore_axis_name="s")

    @functools.partial(
        pl.kernel,
        mesh=mesh,
        out_type=jax.ShapeDtypeStruct((3 * c * K * m,), jnp.float32),
        scratch_types=[
            pltpu.VMEM((ppw,), jnp.int32),
            pltpu.VMEM((ppw,), jnp.int32),
            pltpu.VMEM((ppw,), jnp.int32),
            pltpu.VMEM((ppw,), jnp.int32),
            pltpu.VMEM((c * ppw,), jnp.float32),
            pltpu.VMEM((c * ppw,), jnp.float32),
            pltpu.VMEM((3 * c * K * ppw,), jnp.float32),
            pltpu.SemaphoreType.DMA,
        ],
    )
    def sc_build(x_hbm, idx_hbm, f_hbm, idxk_v, g0_v, g1_v, g2_v,
                 p_v, q_v, f_v, sem):
        gidx = (g0_v, g1_v, g2_v)
        wid = lax.axis_index("s") * _NC + lax.axis_index("c")
        bb = wid // wpb
        base_pt = (wid % wpb) * ppw
        gbase = bb * n + base_pt

        for cc in range(c):
            pltpu.sync_copy(x_hbm.at[pl.ds((bb * c + cc) * n + base_pt, ppw)],
                            p_v.at[pl.ds(cc * ppw, ppw)])

        for k in range(K):
            pltpu.sync_copy(idx_hbm.at[bb * K + k, 0, pl.ds(base_pt, ppw)],
                            idxk_v)

            def ibody(j, _):
                off = j * _L
                nbr = idxk_v[pl.ds(off, _L)]
                for cc in range(c):
                    gidx[cc][pl.ds(off, _L)] = nbr + (bb * c + cc) * n
                return 0

            lax.fori_loop(0, ppw // _L, ibody, 0)

            cps = [pltpu.async_copy(x_hbm.at[gidx[cc]],
                                    q_v.at[pl.ds(cc * ppw, ppw)], sem)
                   for cc in range(c)]
            for cp in cps:
                cp.wait()

            def jbody(j, _, k=k):
                off = j * _L
                for cc in range(c):
                    p = p_v[pl.ds(cc * ppw + off, _L)]
                    q = q_v[pl.ds(cc * ppw + off, _L)]
                    f_v[pl.ds((cc * K + k) * ppw + off, _L)] = p
                    f_v[pl.ds(((c + cc) * K + k) * ppw + off, _L)] = q
                    f_v[pl.ds(((2 * c + cc) * K + k) * ppw + off, _L)] = p - q
                return 0

            lax.fori_loop(0, ppw // _L, jbody, 0)

        for cc in range(3 * c):
            cps = [pltpu.async_copy(
                       f_v.at[pl.ds((cc * K + k) * ppw, ppw)],
                       f_hbm.at[pl.ds((cc * K + k) * m + gbase, ppw)], sem)
                   for k in range(K)]
            for cp in cps:
                cp.wait()

    return sc_build(xyz_flat, idx_r).reshape(3 * c, K, m)


def _bn_scale_shift(w, sum_ref, s_ref, g_ref, b_ref, e):
    mu = sum_ref[...] / e
    m2 = s_ref[...] / e
    hp = jax.lax.Precision.HIGHEST
    mean = lax.dot_general(w, mu, (((1,), (0,)), ((), ())),
                           preferred_element_type=jnp.float32, precision=hp)
    wm = lax.dot_general(w, m2, (((1,), (0,)), ((), ())),
                         preferred_element_type=jnp.float32, precision=hp)
    ey2 = jnp.sum(wm * w, axis=1, keepdims=True)
    var = ey2 - mean * mean
    scale = g_ref[...] / jnp.sqrt(var + EPS)
    shift = b_ref[...] - mean * scale
    return scale, shift


def _x1_moments_body(f_ref, w_ref, g_ref, b_ref,
                     sumf_ref, sf_ref, sum_ref, s_ref, sc_v, sh_v,
                     *, e, mt):
    i = pl.program_id(0)

    @pl.when(i == 0)
    def _():
        sumf_ref[...] = jnp.zeros_like(sumf_ref)
        sf_ref[...] = jnp.zeros_like(sf_ref)
        sum_ref[...] = jnp.zeros_like(sum_ref)
        s_ref[...] = jnp.zeros_like(s_ref)

    fall = jnp.concatenate([f_ref[:, k, :] for k in range(K)], axis=1)

    @pl.when(i < mt)
    def _():
        sumf_ref[...] += jnp.sum(fall, axis=1, keepdims=True)
        sf_ref[...] += lax.dot_general(fall, fall, (((1,), (1,)), ((), ())),
                                       preferred_element_type=jnp.float32)

    @pl.when(i == mt)
    def _():
        scale, shift = _bn_scale_shift(w_ref[...], sumf_ref, sf_ref,
                                       g_ref, b_ref, e)
        sc_v[...] = scale
        sh_v[...] = shift

    @pl.when(i >= mt)
    def _():
        w = w_ref[...]
        y = lax.dot_general(w, fall, (((1,), (0,)), ((), ())),
                            preferred_element_type=jnp.float32)
        x1 = _lrelu(y * sc_v[...] + sh_v[...])
        sum_ref[...] += jnp.sum(x1, axis=1, keepdims=True)
        s_ref[...] += lax.dot_general(x1, x1, (((1,), (1,)), ((), ())),
                                      preferred_element_type=jnp.float32)


def _x1_moments(f, w1, g1, b1, *, e, tile=2048):
    d, _, m = f.shape
    co = w1.shape[0]
    mt = m // tile
    grid = (2 * mt,)
    return pl.pallas_call(
        functools.partial(_x1_moments_body, e=e, mt=mt),
        grid=grid,
        in_specs=[
            pl.BlockSpec((d, K, tile), lambda i: (0, 0, i % mt)),
            pl.BlockSpec((co, d), lambda i: (0, 0)),
            pl.BlockSpec((co, 1), lambda i: (0, 0)),
            pl.BlockSpec((co, 1), lambda i: (0, 0)),
        ],
        out_specs=[
            pl.BlockSpec((d, 1), lambda i: (0, 0)),
            pl.BlockSpec((d, d), lambda i: (0, 0)),
            pl.BlockSpec((co, 1), lambda i: (0, 0)),
            pl.BlockSpec((co, co), lambda i: (0, 0)),
        ],
        out_shape=[
            jax.ShapeDtypeStruct((d, 1), jnp.float32),
            jax.ShapeDtypeStruct((d, d), jnp.float32),
            jax.ShapeDtypeStruct((co, 1), jnp.float32),
            jax.ShapeDtypeStruct((co, co), jnp.float32),
        ],
        scratch_shapes=[
            pltpu.VMEM((co, 1), jnp.float32),
            pltpu.VMEM((co, 1), jnp.float32),
        ],
    )(f, w1, g1, b1)


def _final_body(f_ref, w1_ref, sumf_ref, sf_ref, g1_ref, b1_ref,
                w2_ref, sumx_ref, sx_ref, g2_ref, b2_ref, out_ref,
                sc1_v, sh1_v, sc2_v, sh2_v, *, e):
    @pl.when(jnp.logical_and(pl.program_id(0) == 0, pl.program_id(1) == 0))
    def _():
        s1, h1 = _bn_scale_shift(w1_ref[...], sumf_ref, sf_ref,
                                 g1_ref, b1_ref, e)
        sc1_v[...] = s1
        sh1_v[...] = h1
        s2, h2 = _bn_scale_shift(w2_ref[...], sumx_ref, sx_ref,
                                 g2_ref, b2_ref, e)
        sc2_v[...] = s2
        sh2_v[...] = h2

    w1 = w1_ref[...]
    sc1 = sc1_v[...]
    sh1 = sh1_v[...]
    w2 = w2_ref[...]
    sc2 = sc2_v[...]
    sh2 = sh2_v[...]
    tile = f_ref.shape[2]
    fall = jnp.concatenate([f_ref[:, k, :] for k in range(K)], axis=1)
    y1 = lax.dot_general(w1, fall, (((1,), (0,)), ((), ())),
                         preferred_element_type=jnp.float32)
    x1 = _lrelu(y1 * sc1 + sh1)
    y2 = lax.dot_general(w2, x1, (((1,), (0,)), ((), ())),
                         preferred_element_type=jnp.float32)
    x2 = _lrelu(y2 * sc2 + sh2)
    acc = x2[:, :tile]
    for k in range(1, K):
        acc = jnp.maximum(acc, x2[:, k * tile:(k + 1) * tile])
    out_ref[0] = acc


def _final(f, w1, sum_f, s_f, g1, b1, w2, sum_x, s_x, g2, b2,
           *, e, b, n, tile=512):
    d, _, m = f.shape
    co = w2.shape[0]
    nt = n // tile
    grid = (b, nt)
    return pl.pallas_call(
        functools.partial(_final_body, e=e),
        grid=grid,
        in_specs=[
            pl.BlockSpec((d, K, tile), lambda i, j: (0, 0, i * nt + j)),
            pl.BlockSpec((co, d), lambda i, j: (0, 0)),
            pl.BlockSpec((d, 1), lambda i, j: (0, 0)),
            pl.BlockSpec((d, d), lambda i, j: (0, 0)),
            pl.BlockSpec((co, 1), lambda i, j: (0, 0)),
            pl.BlockSpec((co, 1), lambda i, j: (0, 0)),
            pl.BlockSpec((co, co), lambda i, j: (0, 0)),
            pl.BlockSpec((co, 1), lambda i, j: (0, 0)),
            pl.BlockSpec((co, co), lambda i, j: (0, 0)),
            pl.BlockSpec((co, 1), lambda i, j: (0, 0)),
            pl.BlockSpec((co, 1), lambda i, j: (0, 0)),
        ],
        out_specs=pl.BlockSpec((1, co, tile), lambda i, j: (i, 0, j)),
        out_shape=jax.ShapeDtypeStruct((b, co, n), jnp.float32),
        scratch_shapes=[
            pltpu.VMEM((co, 1), jnp.float32),
            pltpu.VMEM((co, 1), jnp.float32),
            pltpu.VMEM((co, 1), jnp.float32),
            pltpu.VMEM((co, 1), jnp.float32),
        ],
    )(f, w1, sum_f, s_f, g1, b1, w2, sum_x, s_x, g2, b2)


def kernel(xyz, W1, g1, b1, W2, g2, b2):
    b, c, n = xyz.shape
    e = b * n * K

    idx = _knn(xyz)
    f = _build_edge_features(xyz, idx)
    g1c, b1c = g1[:, None], b1[:, None]
    g2c, b2c = g2[:, None], b2[:, None]
    sum_f, s_f, sum_x, s_x = _x1_moments(f, W1, g1c, b1c, e=e)
    return _final(f, W1, sum_f, s_f, g1c, b1c,
                  W2, sum_x, s_x, g2c, b2c, e=e, b=b, n=n)
